# baseline (speedup 1.0000x reference)
import functools

import jax
import jax.numpy as jnp
from jax.experimental import pallas as pl
from jax.experimental.pallas import tpu as pltpu

_BLK_H = 128
_N_STREAMS = 4


def _body(tc_ref, *refs, b_total, n_total, blk_h, n_streams):
    jb = pl.program_id(0)

    f_refs = refs[:n_streams]
    mem_refs = refs[n_streams:2 * n_streams]
    cen_ref, out_ref, ld_ref, acc_ref, obuf_ref, sem = refs[2 * n_streams:]

    cen = cen_ref[...]
    c2 = jnp.sum(cen * cen, axis=1, keepdims=True)
    tc = tc_ref[0, 0]

    slot = jax.lax.rem(jb, 2)
    sub = blk_h // n_streams

    def copy(s, dst_b, j):
        return pltpu.make_async_copy(
            obuf_ref.at[s, :, pl.ds(j * sub, sub), :],
            out_ref.at[dst_b, :, pl.ds(j * sub, sub), :],
            sem.at[s, j])

    @pl.when(jb >= 2)
    def _drain():
        for j in range(n_streams):
            copy(slot, jb, j).wait()

    sum_a = 0.0
    cnt_a = 0.0
    sum_all = 0.0
    for half, fref in enumerate(f_refs):
        fblk = fref[0]
        f2 = jnp.sum(fblk * fblk, axis=0)
        ft = jnp.transpose(fblk, (1, 0, 2))

        rows = []
        for hh in range(sub):
            cross = jax.lax.dot_general(
                cen, ft[hh], (((1,), (0,)), ((), ())),
                preferred_element_type=jnp.float32)
            g = c2 - 2.0 * cross
            rows.append(jnp.min(g, axis=0, keepdims=True))
        gmin = jnp.concatenate(rows, axis=0)

        dmin = jnp.sqrt(jnp.maximum(f2 + gmin, 1e-12))
        mask = dmin > tc
        obuf_ref[slot, :, half * sub:(half + 1) * sub, :] = jnp.where(
            mask[None], mem_refs[half][...], fblk)
        copy(slot, jb, half).start()

        sum_a = sum_a + jnp.sum(jnp.where(mask, dmin, 0.0))
        cnt_a = cnt_a + jnp.sum(mask.astype(jnp.float32))
        sum_all = sum_all + jnp.sum(dmin)

    @pl.when(jb == 0)
    def _init():
        acc_ref[0] = 0.0
        acc_ref[1] = 0.0
        acc_ref[2] = 0.0

    acc_ref[0] += sum_a
    acc_ref[1] += cnt_a
    acc_ref[2] += sum_all

    @pl.when(jb == b_total - 1)
    def _fin():
        for j in range(n_streams):
            copy(1 - slot, jb, j).wait()
        for j in range(n_streams):
            copy(slot, jb, j).wait()
        sa = acc_ref[0]
        ca = acc_ref[1]
        sn = acc_ref[2] - sa
        cn = jnp.float32(n_total) - ca
        mean_ano = sa / jnp.maximum(ca, 1.0)
        mean_nor = sn / jnp.maximum(cn, 1.0)
        ld_ref[0, 0] = jnp.where(ca > 0.0, mean_nor / (mean_ano + 0.001),
                                 mean_nor)


def kernel(f, center, Tc, memory, is_object):
    b, c, h, w = f.shape
    k = center.shape[0]
    mem_r = memory.reshape(c, h, w)
    tc_arr = jnp.asarray(Tc, dtype=jnp.float32).reshape(1, 1)

    ns = _N_STREAMS
    sub = _BLK_H // ns
    body = functools.partial(_body, b_total=b, n_total=b * h * w,
                             blk_h=_BLK_H, n_streams=ns)
    f_specs = [
        pl.BlockSpec((1, c, sub, w),
                     functools.partial(lambda s, jb: (jb, 0, s, 0), s))
        for s in range(ns)
    ]
    mem_specs = [
        pl.BlockSpec((c, sub, w),
                     functools.partial(lambda s, jb: (0, s, 0), s))
        for s in range(ns)
    ]
    out, ld = pl.pallas_call(
        body,
        grid=(b,),
        in_specs=[
            pl.BlockSpec(memory_space=pltpu.SMEM),
            *f_specs,
            *mem_specs,
            pl.BlockSpec((k, c), lambda jb: (0, 0)),
        ],
        out_specs=[
            pl.BlockSpec(memory_space=pltpu.MemorySpace.HBM),
            pl.BlockSpec(memory_space=pltpu.SMEM),
        ],
        out_shape=[
            jax.ShapeDtypeStruct((b, c, h, w), jnp.float32),
            jax.ShapeDtypeStruct((1, 1), jnp.float32),
        ],
        scratch_shapes=[
            pltpu.SMEM((4,), jnp.float32),
            pltpu.VMEM((2, c, _BLK_H, w), jnp.float32),
            pltpu.SemaphoreType.DMA((2, _N_STREAMS)),
        ],
        compiler_params=pltpu.CompilerParams(
            dimension_semantics=("arbitrary",)),
    )(tc_arr, *([f] * ns), *([mem_r] * ns), center)

    return out, ld[0, 0]


# speedup vs baseline: 1.0477x; 1.0477x over previous
import functools

import jax
import jax.numpy as jnp
from jax.experimental import pallas as pl
from jax.experimental.pallas import tpu as pltpu

_BLK_H = 128
_N_STREAMS = 8


def _body(tc_ref, *refs, b_total, n_total, blk_h, n_streams):
    jb = pl.program_id(0)

    f_refs = refs[:n_streams]
    mem_refs = refs[n_streams:2 * n_streams]
    cen_ref, out_ref, ld_ref, acc_ref, obuf_ref, sem = refs[2 * n_streams:]

    cen = cen_ref[...]
    c2 = jnp.sum(cen * cen, axis=1, keepdims=True)
    tc = tc_ref[0, 0]

    slot = jax.lax.rem(jb, 3)
    sub = blk_h // n_streams

    def copy(s, dst_b, j):
        return pltpu.make_async_copy(
            obuf_ref.at[s, :, pl.ds(j * sub, sub), :],
            out_ref.at[dst_b, :, pl.ds(j * sub, sub), :],
            sem.at[s, j])

    @pl.when(jb >= 3)
    def _drain():
        for j in range(n_streams):
            copy(slot, jb, j).wait()

    sum_a = 0.0
    cnt_a = 0.0
    sum_all = 0.0
    for half, fref in enumerate(f_refs):
        fblk = fref[0]
        f2 = jnp.sum(fblk * fblk, axis=0)
        ft = jnp.transpose(fblk, (1, 0, 2))

        rows = []
        for hh in range(sub):
            cross = jax.lax.dot_general(
                cen, ft[hh], (((1,), (0,)), ((), ())),
                preferred_element_type=jnp.float32)
            g = c2 - 2.0 * cross
            rows.append(jnp.min(g, axis=0, keepdims=True))
        gmin = jnp.concatenate(rows, axis=0)

        dmin = jnp.sqrt(jnp.maximum(f2 + gmin, 1e-12))
        mask = dmin > tc
        obuf_ref[slot, :, half * sub:(half + 1) * sub, :] = jnp.where(
            mask[None], mem_refs[half][...], fblk)
        copy(slot, jb, half).start()

        sum_a = sum_a + jnp.sum(jnp.where(mask, dmin, 0.0))
        cnt_a = cnt_a + jnp.sum(mask.astype(jnp.float32))
        sum_all = sum_all + jnp.sum(dmin)

    @pl.when(jb == 0)
    def _init():
        acc_ref[0] = 0.0
        acc_ref[1] = 0.0
        acc_ref[2] = 0.0

    acc_ref[0] += sum_a
    acc_ref[1] += cnt_a
    acc_ref[2] += sum_all

    @pl.when(jb == b_total - 1)
    def _fin():
        prev1 = jax.lax.rem(jb + 2, 3)
        prev2 = jax.lax.rem(jb + 1, 3)
        for j in range(n_streams):
            copy(prev2, jb, j).wait()
        for j in range(n_streams):
            copy(prev1, jb, j).wait()
        for j in range(n_streams):
            copy(slot, jb, j).wait()
        sa = acc_ref[0]
        ca = acc_ref[1]
        sn = acc_ref[2] - sa
        cn = jnp.float32(n_total) - ca
        mean_ano = sa / jnp.maximum(ca, 1.0)
        mean_nor = sn / jnp.maximum(cn, 1.0)
        ld_ref[0, 0] = jnp.where(ca > 0.0, mean_nor / (mean_ano + 0.001),
                                 mean_nor)


def kernel(f, center, Tc, memory, is_object):
    b, c, h, w = f.shape
    k = center.shape[0]
    mem_r = memory.reshape(c, h, w)
    tc_arr = jnp.asarray(Tc, dtype=jnp.float32).reshape(1, 1)

    ns = _N_STREAMS
    sub = _BLK_H // ns
    body = functools.partial(_body, b_total=b, n_total=b * h * w,
                             blk_h=_BLK_H, n_streams=ns)
    f_specs = [
        pl.BlockSpec((1, c, sub, w),
                     functools.partial(lambda s, jb: (jb, 0, s, 0), s))
        for s in range(ns)
    ]
    mem_specs = [
        pl.BlockSpec((c, sub, w),
                     functools.partial(lambda s, jb: (0, s, 0), s))
        for s in range(ns)
    ]
    out, ld = pl.pallas_call(
        body,
        grid=(b,),
        in_specs=[
            pl.BlockSpec(memory_space=pltpu.SMEM),
            *f_specs,
            *mem_specs,
            pl.BlockSpec((k, c), lambda jb: (0, 0)),
        ],
        out_specs=[
            pl.BlockSpec(memory_space=pltpu.MemorySpace.HBM),
            pl.BlockSpec(memory_space=pltpu.SMEM),
        ],
        out_shape=[
            jax.ShapeDtypeStruct((b, c, h, w), jnp.float32),
            jax.ShapeDtypeStruct((1, 1), jnp.float32),
        ],
        scratch_shapes=[
            pltpu.SMEM((4,), jnp.float32),
            pltpu.VMEM((3, c, _BLK_H, w), jnp.float32),
            pltpu.SemaphoreType.DMA((3, _N_STREAMS)),
        ],
        compiler_params=pltpu.CompilerParams(
            dimension_semantics=("arbitrary",)),
    )(tc_arr, *([f] * ns), *([mem_r] * ns), center)

    return out, ld[0, 0]
